# Initial kernel scaffold; baseline (speedup 1.0000x reference)
import jax
import jax.numpy as jnp
from jax.experimental import pallas as pl


def kernel(x, edge_index, Wl_i, Wr_i, b_i, Wl_f, Wr_f, b_f, Wl_c, Wr_c, b_c, Wl_o, Wr_o, b_o):
    raise NotImplementedError("write your pallas kernel here")


# speedup vs baseline: 5.2025x; 5.2025x over previous
import jax
import jax.numpy as jnp
from jax import lax
from jax.experimental import pallas as pl
from jax.experimental.pallas import tpu as pltpu
from jax.experimental.pallas import tpu_sc as plsc

N = 10000
E = 320000
D = 128
OUT = 128

NC = 2
NS = 16
NW = NC * NS

EDGES_PER_W = E // NW
CHUNK = 64
FULL_CHUNKS = EDGES_PER_W // CHUNK
TAIL = EDGES_PER_W - FULL_CHUNKS * CHUNK
NP = 10240
ROWS_PER_TILE = NP // NS
HR = NP // 128


def _sc_body(x_hbm, src_hbm, dst_hbm, zrow_hbm,
             agg_hbm, hist_hbm,
             src_v, dst_v, rows_v, src_t, dst_t, rows_t, hist_v,
             acc_sh):
    c = lax.axis_index("c")
    s = lax.axis_index("s")
    w = c * NS + s
    base = w * EDGES_PER_W
    row0 = s * ROWS_PER_TILE

    pltpu.sync_copy(zrow_hbm, rows_v)
    for k in range(ROWS_PER_TILE // CHUNK):
        pltpu.sync_copy(rows_v, acc_sh.at[pl.ds(row0 + k * CHUNK, CHUNK), :])

    zeros16 = jnp.zeros((16,), jnp.float32)

    def zstep(i, carry):
        for g in range(128 // 16):
            hist_v[i, pl.ds(g * 16, 16)] = zeros16
        return carry

    lax.fori_loop(0, HR, zstep, 0)
    plsc.subcore_barrier()

    ones16 = jnp.ones((16,), jnp.float32)

    def step(j, carry):
        off = base + j * CHUNK
        pltpu.sync_copy(src_hbm.at[pl.ds(off, CHUNK)], src_v)
        pltpu.sync_copy(dst_hbm.at[pl.ds(off, CHUNK)], dst_v)
        pltpu.sync_copy(x_hbm.at[src_v], rows_v)
        pltpu.sync_copy(rows_v, acc_sh.at[dst_v], add=True)
        for g in range(CHUNK // 16):
            ig = dst_v[pl.ds(g * 16, 16)]
            plsc.addupdate_scatter(
                hist_v, [lax.shift_right_logical(ig, 7),
                         lax.bitwise_and(ig, 127)], ones16)
        return carry

    lax.fori_loop(0, FULL_CHUNKS, step, 0)

    off = base + FULL_CHUNKS * CHUNK
    pltpu.sync_copy(src_hbm.at[pl.ds(off, TAIL)], src_t)
    pltpu.sync_copy(dst_hbm.at[pl.ds(off, TAIL)], dst_t)
    pltpu.sync_copy(x_hbm.at[src_t], rows_t)
    pltpu.sync_copy(rows_t, acc_sh.at[dst_t], add=True)
    it = dst_t[...]
    plsc.addupdate_scatter(
        hist_v, [lax.shift_right_logical(it, 7),
                 lax.bitwise_and(it, 127)], ones16)

    plsc.subcore_barrier()

    for k in range(ROWS_PER_TILE // CHUNK):
        r = row0 + k * CHUNK
        pltpu.sync_copy(acc_sh.at[pl.ds(r, CHUNK), :], rows_v)
        pltpu.sync_copy(rows_v, agg_hbm.at[c, pl.ds(r, CHUNK), :])
    pltpu.sync_copy(hist_v, hist_hbm.at[w])


@jax.jit
def _sc_segment_sum(x, src, dst):
    zrow = jnp.zeros((CHUNK, D), jnp.float32)
    mesh = plsc.VectorSubcoreMesh(core_axis_name="c", subcore_axis_name="s",
                                  num_cores=NC, num_subcores=NS)
    fn = pl.kernel(
        _sc_body,
        out_type=(jax.ShapeDtypeStruct((NC, NP, D), jnp.float32),
                  jax.ShapeDtypeStruct((NW, HR, 128), jnp.float32)),
        mesh=mesh,
        compiler_params=pltpu.CompilerParams(needs_layout_passes=False),
        scratch_types=[
            pltpu.VMEM((CHUNK,), jnp.int32),
            pltpu.VMEM((CHUNK,), jnp.int32),
            pltpu.VMEM((CHUNK, D), jnp.float32),
            pltpu.VMEM((TAIL,), jnp.int32),
            pltpu.VMEM((TAIL,), jnp.int32),
            pltpu.VMEM((TAIL, D), jnp.float32),
            pltpu.VMEM((HR, 128), jnp.float32),
            pltpu.VMEM_SHARED((NP, D), jnp.float32),
        ],
    )
    return fn(x, src, dst, zrow)


def _tc_body(x_ref, a_ref, h_ref, wl_ref, wr_ref, b_ref, h_out, c_out):
    cnt_row = jnp.sum(h_ref[...], axis=0, keepdims=True)
    inv_row = 1.0 / jnp.maximum(cnt_row, 1.0)
    inv_col = lax.dot_general(inv_row, jnp.ones((1, D), jnp.float32),
                              ((( 0,), (0,)), ((), ())),
                              precision=lax.Precision.HIGHEST)
    agg = (a_ref[0] + a_ref[1]) * inv_col
    z = (jnp.dot(agg, wl_ref[...], preferred_element_type=jnp.float32,
                 precision=lax.Precision.HIGHEST)
         + jnp.dot(x_ref[...], wr_ref[...], preferred_element_type=jnp.float32,
                   precision=lax.Precision.HIGHEST)
         + b_ref[...])
    i_g = jax.nn.sigmoid(z[:, :OUT])
    t_g = jnp.tanh(z[:, OUT:2 * OUT])
    o_g = jax.nn.sigmoid(z[:, 2 * OUT:])
    c_new = i_g * t_g
    h_out[...] = o_g * jnp.tanh(c_new)
    c_out[...] = c_new


BLK = 1024
HRB = BLK // 128


@jax.jit
def _tc_gates(xp, agg2, hist, wl3, wr3, b3):
    grid = (NP // BLK,)
    return pl.pallas_call(
        _tc_body,
        grid=grid,
        in_specs=[
            pl.BlockSpec((BLK, D), lambda i: (i, 0)),
            pl.BlockSpec((NC, BLK, D), lambda i: (0, i, 0)),
            pl.BlockSpec((NW, BLK), lambda i: (0, i)),
            pl.BlockSpec((D, 3 * OUT), lambda i: (0, 0)),
            pl.BlockSpec((D, 3 * OUT), lambda i: (0, 0)),
            pl.BlockSpec((1, 3 * OUT), lambda i: (0, 0)),
        ],
        out_specs=[
            pl.BlockSpec((BLK, OUT), lambda i: (i, 0)),
            pl.BlockSpec((BLK, OUT), lambda i: (i, 0)),
        ],
        out_shape=[
            jax.ShapeDtypeStruct((NP, OUT), jnp.float32),
            jax.ShapeDtypeStruct((NP, OUT), jnp.float32),
        ],
    )(xp, agg2, hist, wl3, wr3, b3)


def kernel(x, edge_index, Wl_i, Wr_i, b_i, Wl_f, Wr_f, b_f,
           Wl_c, Wr_c, b_c, Wl_o, Wr_o, b_o):
    agg2, hist = _sc_segment_sum(x, edge_index[0], edge_index[1])
    hist = hist.reshape(NW, NP)
    xp = jnp.concatenate([x, jnp.zeros((NP - N, D), x.dtype)], axis=0)
    wl3 = jnp.concatenate([Wl_i, Wl_c, Wl_o], axis=1)
    wr3 = jnp.concatenate([Wr_i, Wr_c, Wr_o], axis=1)
    b3 = jnp.concatenate([b_i, b_c, b_o]).reshape(1, 3 * OUT)
    h_new, c_new = _tc_gates(xp, agg2, hist, wl3, wr3, b3)
    return (h_new[:N], c_new[:N])


# speedup vs baseline: 8.2424x; 1.5843x over previous
import jax
import jax.numpy as jnp
from jax import lax
from jax.experimental import pallas as pl
from jax.experimental.pallas import tpu as pltpu
from jax.experimental.pallas import tpu_sc as plsc

N = 10000
E = 320000
D = 128
OUT = 128

NC = 2
NS = 16
NW = NC * NS

EDGES_PER_W = E // NW
CHUNK = 64
FULL_CHUNKS = EDGES_PER_W // CHUNK
TAIL = EDGES_PER_W - FULL_CHUNKS * CHUNK
NP = 10240
ROWS_PER_TILE = NP // NS
HR = NP // 128


def _sc_body(x_hbm, src_hbm, dst_hbm, zrow_hbm,
             agg_hbm, hist_hbm,
             srcA, dstA, rowsA, srcB, dstB, rowsB,
             src_t, dst_t, rows_t, hist_v,
             acc_sh,
             semiA, semiB, semgA, semgB, semsA, semsB):
    c = lax.axis_index("c")
    s = lax.axis_index("s")
    w = c * NS + s
    base = w * EDGES_PER_W
    row0 = s * ROWS_PER_TILE

    pltpu.sync_copy(zrow_hbm, rowsA)
    for k in range(ROWS_PER_TILE // CHUNK):
        pltpu.sync_copy(rowsA, acc_sh.at[pl.ds(row0 + k * CHUNK, CHUNK), :])

    zeros16 = jnp.zeros((16,), jnp.float32)

    def zstep(i, carry):
        for g in range(128 // 16):
            hist_v[i, pl.ds(g * 16, 16)] = zeros16
        return carry

    lax.fori_loop(0, HR, zstep, 0)
    plsc.subcore_barrier()

    ones16 = jnp.ones((16,), jnp.float32)

    def hist_update(dref):
        for g in range(CHUNK // 16):
            ig = dref[pl.ds(g * 16, 16)]
            plsc.addupdate_scatter(
                hist_v, [lax.shift_right_logical(ig, 7),
                         lax.bitwise_and(ig, 127)], ones16)

    def step(i, carry):
        offA = base + (2 * i) * CHUNK
        offB = offA + CHUNK
        ia = pltpu.async_copy(src_hbm.at[pl.ds(offA, CHUNK)], srcA, semiA)
        ib = pltpu.async_copy(dst_hbm.at[pl.ds(offA, CHUNK)], dstA, semiA)
        ic = pltpu.async_copy(src_hbm.at[pl.ds(offB, CHUNK)], srcB, semiB)
        id_ = pltpu.async_copy(dst_hbm.at[pl.ds(offB, CHUNK)], dstB, semiB)
        ia.wait()
        ib.wait()
        ga = pltpu.async_copy(x_hbm.at[srcA], rowsA, semgA)
        ic.wait()
        id_.wait()
        gb = pltpu.async_copy(x_hbm.at[srcB], rowsB, semgB)
        ga.wait()
        sa = pltpu.async_copy(rowsA, acc_sh.at[dstA], semsA, add=True)
        hist_update(dstA)
        gb.wait()
        sb = pltpu.async_copy(rowsB, acc_sh.at[dstB], semsB, add=True)
        hist_update(dstB)
        sa.wait()
        sb.wait()
        return carry

    lax.fori_loop(0, FULL_CHUNKS // 2, step, 0)

    off = base + FULL_CHUNKS * CHUNK
    pltpu.sync_copy(src_hbm.at[pl.ds(off, TAIL)], src_t)
    pltpu.sync_copy(dst_hbm.at[pl.ds(off, TAIL)], dst_t)
    pltpu.sync_copy(x_hbm.at[src_t], rows_t)
    pltpu.sync_copy(rows_t, acc_sh.at[dst_t], add=True)
    it = dst_t[...]
    plsc.addupdate_scatter(
        hist_v, [lax.shift_right_logical(it, 7),
                 lax.bitwise_and(it, 127)], ones16)

    plsc.subcore_barrier()

    for k in range(ROWS_PER_TILE // CHUNK):
        r = row0 + k * CHUNK
        pltpu.sync_copy(acc_sh.at[pl.ds(r, CHUNK), :], rowsA)
        pltpu.sync_copy(rowsA, agg_hbm.at[c, pl.ds(r, CHUNK), :])
    pltpu.sync_copy(hist_v, hist_hbm.at[w])


@jax.jit
def _sc_segment_sum(x, src, dst):
    zrow = jnp.zeros((CHUNK, D), jnp.float32)
    mesh = plsc.VectorSubcoreMesh(core_axis_name="c", subcore_axis_name="s",
                                  num_cores=NC, num_subcores=NS)
    fn = pl.kernel(
        _sc_body,
        out_type=(jax.ShapeDtypeStruct((NC, NP, D), jnp.float32),
                  jax.ShapeDtypeStruct((NW, HR, 128), jnp.float32)),
        mesh=mesh,
        compiler_params=pltpu.CompilerParams(needs_layout_passes=False),
        scratch_types=[
            pltpu.VMEM((CHUNK,), jnp.int32),
            pltpu.VMEM((CHUNK,), jnp.int32),
            pltpu.VMEM((CHUNK, D), jnp.float32),
            pltpu.VMEM((CHUNK,), jnp.int32),
            pltpu.VMEM((CHUNK,), jnp.int32),
            pltpu.VMEM((CHUNK, D), jnp.float32),
            pltpu.VMEM((TAIL,), jnp.int32),
            pltpu.VMEM((TAIL,), jnp.int32),
            pltpu.VMEM((TAIL, D), jnp.float32),
            pltpu.VMEM((HR, 128), jnp.float32),
            pltpu.VMEM_SHARED((NP, D), jnp.float32),
            pltpu.SemaphoreType.DMA,
            pltpu.SemaphoreType.DMA,
            pltpu.SemaphoreType.DMA,
            pltpu.SemaphoreType.DMA,
            pltpu.SemaphoreType.DMA,
            pltpu.SemaphoreType.DMA,
        ],
    )
    return fn(x, src, dst, zrow)


def _tc_body(x_ref, a_ref, h_ref, wl_ref, wr_ref, b_ref, h_out, c_out):
    cnt_row = jnp.sum(h_ref[...], axis=0, keepdims=True)
    inv_row = 1.0 / jnp.maximum(cnt_row, 1.0)
    inv_col = lax.dot_general(inv_row, jnp.ones((1, D), jnp.float32),
                              ((( 0,), (0,)), ((), ())),
                              precision=lax.Precision.HIGHEST)
    agg = (a_ref[0] + a_ref[1]) * inv_col
    z = (jnp.dot(agg, wl_ref[...], preferred_element_type=jnp.float32,
                 precision=lax.Precision.HIGHEST)
         + jnp.dot(x_ref[...], wr_ref[...], preferred_element_type=jnp.float32,
                   precision=lax.Precision.HIGHEST)
         + b_ref[...])
    i_g = jax.nn.sigmoid(z[:, :OUT])
    t_g = jnp.tanh(z[:, OUT:2 * OUT])
    o_g = jax.nn.sigmoid(z[:, 2 * OUT:])
    c_new = i_g * t_g
    h_out[...] = o_g * jnp.tanh(c_new)
    c_out[...] = c_new


BLK = 1024
HRB = BLK // 128


@jax.jit
def _tc_gates(xp, agg2, hist, wl3, wr3, b3):
    grid = (NP // BLK,)
    return pl.pallas_call(
        _tc_body,
        grid=grid,
        in_specs=[
            pl.BlockSpec((BLK, D), lambda i: (i, 0)),
            pl.BlockSpec((NC, BLK, D), lambda i: (0, i, 0)),
            pl.BlockSpec((NW, BLK), lambda i: (0, i)),
            pl.BlockSpec((D, 3 * OUT), lambda i: (0, 0)),
            pl.BlockSpec((D, 3 * OUT), lambda i: (0, 0)),
            pl.BlockSpec((1, 3 * OUT), lambda i: (0, 0)),
        ],
        out_specs=[
            pl.BlockSpec((BLK, OUT), lambda i: (i, 0)),
            pl.BlockSpec((BLK, OUT), lambda i: (i, 0)),
        ],
        out_shape=[
            jax.ShapeDtypeStruct((NP, OUT), jnp.float32),
            jax.ShapeDtypeStruct((NP, OUT), jnp.float32),
        ],
    )(xp, agg2, hist, wl3, wr3, b3)


def kernel(x, edge_index, Wl_i, Wr_i, b_i, Wl_f, Wr_f, b_f,
           Wl_c, Wr_c, b_c, Wl_o, Wr_o, b_o):
    agg2, hist = _sc_segment_sum(x, edge_index[0], edge_index[1])
    hist = hist.reshape(NW, NP)
    xp = jnp.concatenate([x, jnp.zeros((NP - N, D), x.dtype)], axis=0)
    wl3 = jnp.concatenate([Wl_i, Wl_c, Wl_o], axis=1)
    wr3 = jnp.concatenate([Wr_i, Wr_c, Wr_o], axis=1)
    b3 = jnp.concatenate([b_i, b_c, b_o]).reshape(1, 3 * OUT)
    h_new, c_new = _tc_gates(xp, agg2, hist, wl3, wr3, b3)
    return (h_new[:N], c_new[:N])


# speedup vs baseline: 9.3256x; 1.1314x over previous
import jax
import jax.numpy as jnp
from jax import lax
from jax.experimental import pallas as pl
from jax.experimental.pallas import tpu as pltpu
from jax.experimental.pallas import tpu_sc as plsc

N = 10000
E = 320000
D = 128
OUT = 128

NC = 2
NS = 16
NW = NC * NS

EDGES_PER_W = E // NW
CHUNK = 128
FULL_CHUNKS = EDGES_PER_W // CHUNK
TAIL = EDGES_PER_W - FULL_CHUNKS * CHUNK
NP = 10240
ROWS_PER_TILE = NP // NS
HR = NP // 128


def _sc_body(x_hbm, src_hbm, dst_hbm, zrow_hbm,
             agg_hbm, hist_hbm,
             srcA, dstA, rowsA, srcB, dstB, rowsB,
             src_t, dst_t, rows_t, hist_v,
             acc_sh,
             semiA, semiB, semgA, semgB, semsA, semsB):
    c = lax.axis_index("c")
    s = lax.axis_index("s")
    w = c * NS + s
    base = w * EDGES_PER_W
    row0 = s * ROWS_PER_TILE

    pltpu.sync_copy(zrow_hbm, rowsA)
    for k in range(ROWS_PER_TILE // CHUNK):
        pltpu.sync_copy(rowsA, acc_sh.at[pl.ds(row0 + k * CHUNK, CHUNK), :])

    zeros16 = jnp.zeros((16,), jnp.float32)

    def zstep(i, carry):
        for g in range(128 // 16):
            hist_v[i, pl.ds(g * 16, 16)] = zeros16
        return carry

    lax.fori_loop(0, HR, zstep, 0)
    plsc.subcore_barrier()

    ones16 = jnp.ones((16,), jnp.float32)

    def hist_update(dref):
        for g in range(CHUNK // 16):
            ig = dref[pl.ds(g * 16, 16)]
            plsc.addupdate_scatter(
                hist_v, [lax.shift_right_logical(ig, 7),
                         lax.bitwise_and(ig, 127)], ones16)

    def step(i, carry):
        offA = base + (2 * i) * CHUNK
        offB = offA + CHUNK
        ia = pltpu.async_copy(src_hbm.at[pl.ds(offA, CHUNK)], srcA, semiA)
        ib = pltpu.async_copy(dst_hbm.at[pl.ds(offA, CHUNK)], dstA, semiA)
        ic = pltpu.async_copy(src_hbm.at[pl.ds(offB, CHUNK)], srcB, semiB)
        id_ = pltpu.async_copy(dst_hbm.at[pl.ds(offB, CHUNK)], dstB, semiB)
        ia.wait()
        ib.wait()
        ga = pltpu.async_copy(x_hbm.at[srcA], rowsA, semgA)
        ic.wait()
        id_.wait()
        gb = pltpu.async_copy(x_hbm.at[srcB], rowsB, semgB)
        ga.wait()
        sa = pltpu.async_copy(rowsA, acc_sh.at[dstA], semsA, add=True)
        hist_update(dstA)
        gb.wait()
        sb = pltpu.async_copy(rowsB, acc_sh.at[dstB], semsB, add=True)
        hist_update(dstB)
        sa.wait()
        sb.wait()
        return carry

    lax.fori_loop(0, FULL_CHUNKS // 2, step, 0)

    off = base + FULL_CHUNKS * CHUNK
    pltpu.sync_copy(src_hbm.at[pl.ds(off, TAIL)], src_t)
    pltpu.sync_copy(dst_hbm.at[pl.ds(off, TAIL)], dst_t)
    pltpu.sync_copy(x_hbm.at[src_t], rows_t)
    pltpu.sync_copy(rows_t, acc_sh.at[dst_t], add=True)
    it = dst_t[...]
    plsc.addupdate_scatter(
        hist_v, [lax.shift_right_logical(it, 7),
                 lax.bitwise_and(it, 127)], ones16)

    plsc.subcore_barrier()

    for k in range(ROWS_PER_TILE // CHUNK):
        r = row0 + k * CHUNK
        pltpu.sync_copy(acc_sh.at[pl.ds(r, CHUNK), :], rowsA)
        pltpu.sync_copy(rowsA, agg_hbm.at[c, pl.ds(r, CHUNK), :])
    pltpu.sync_copy(hist_v, hist_hbm.at[w])


@jax.jit
def _sc_segment_sum(x, src, dst):
    zrow = jnp.zeros((CHUNK, D), jnp.float32)
    mesh = plsc.VectorSubcoreMesh(core_axis_name="c", subcore_axis_name="s",
                                  num_cores=NC, num_subcores=NS)
    fn = pl.kernel(
        _sc_body,
        out_type=(jax.ShapeDtypeStruct((NC, NP, D), jnp.float32),
                  jax.ShapeDtypeStruct((NW, HR, 128), jnp.float32)),
        mesh=mesh,
        compiler_params=pltpu.CompilerParams(needs_layout_passes=False),
        scratch_types=[
            pltpu.VMEM((CHUNK,), jnp.int32),
            pltpu.VMEM((CHUNK,), jnp.int32),
            pltpu.VMEM((CHUNK, D), jnp.float32),
            pltpu.VMEM((CHUNK,), jnp.int32),
            pltpu.VMEM((CHUNK,), jnp.int32),
            pltpu.VMEM((CHUNK, D), jnp.float32),
            pltpu.VMEM((TAIL,), jnp.int32),
            pltpu.VMEM((TAIL,), jnp.int32),
            pltpu.VMEM((TAIL, D), jnp.float32),
            pltpu.VMEM((HR, 128), jnp.float32),
            pltpu.VMEM_SHARED((NP, D), jnp.float32),
            pltpu.SemaphoreType.DMA,
            pltpu.SemaphoreType.DMA,
            pltpu.SemaphoreType.DMA,
            pltpu.SemaphoreType.DMA,
            pltpu.SemaphoreType.DMA,
            pltpu.SemaphoreType.DMA,
        ],
    )
    return fn(x, src, dst, zrow)


def _tc_body(x_ref, a_ref, h_ref, wl_ref, wr_ref, b_ref, h_out, c_out):
    cnt_row = jnp.sum(h_ref[...], axis=0, keepdims=True)
    inv_row = 1.0 / jnp.maximum(cnt_row, 1.0)
    inv_col = lax.dot_general(inv_row, jnp.ones((1, D), jnp.float32),
                              ((( 0,), (0,)), ((), ())),
                              precision=lax.Precision.HIGHEST)
    agg = (a_ref[0] + a_ref[1]) * inv_col
    z = (jnp.dot(agg, wl_ref[...], preferred_element_type=jnp.float32,
                 precision=lax.Precision.HIGHEST)
         + jnp.dot(x_ref[...], wr_ref[...], preferred_element_type=jnp.float32,
                   precision=lax.Precision.HIGHEST)
         + b_ref[...])
    i_g = jax.nn.sigmoid(z[:, :OUT])
    t_g = jnp.tanh(z[:, OUT:2 * OUT])
    o_g = jax.nn.sigmoid(z[:, 2 * OUT:])
    c_new = i_g * t_g
    h_out[...] = o_g * jnp.tanh(c_new)
    c_out[...] = c_new


BLK = 1024
HRB = BLK // 128


@jax.jit
def _tc_gates(xp, agg2, hist, wl3, wr3, b3):
    grid = (NP // BLK,)
    return pl.pallas_call(
        _tc_body,
        grid=grid,
        in_specs=[
            pl.BlockSpec((BLK, D), lambda i: (i, 0)),
            pl.BlockSpec((NC, BLK, D), lambda i: (0, i, 0)),
            pl.BlockSpec((NW, BLK), lambda i: (0, i)),
            pl.BlockSpec((D, 3 * OUT), lambda i: (0, 0)),
            pl.BlockSpec((D, 3 * OUT), lambda i: (0, 0)),
            pl.BlockSpec((1, 3 * OUT), lambda i: (0, 0)),
        ],
        out_specs=[
            pl.BlockSpec((BLK, OUT), lambda i: (i, 0)),
            pl.BlockSpec((BLK, OUT), lambda i: (i, 0)),
        ],
        out_shape=[
            jax.ShapeDtypeStruct((NP, OUT), jnp.float32),
            jax.ShapeDtypeStruct((NP, OUT), jnp.float32),
        ],
    )(xp, agg2, hist, wl3, wr3, b3)


def kernel(x, edge_index, Wl_i, Wr_i, b_i, Wl_f, Wr_f, b_f,
           Wl_c, Wr_c, b_c, Wl_o, Wr_o, b_o):
    agg2, hist = _sc_segment_sum(x, edge_index[0], edge_index[1])
    hist = hist.reshape(NW, NP)
    xp = jnp.concatenate([x, jnp.zeros((NP - N, D), x.dtype)], axis=0)
    wl3 = jnp.concatenate([Wl_i, Wl_c, Wl_o], axis=1)
    wr3 = jnp.concatenate([Wr_i, Wr_c, Wr_o], axis=1)
    b3 = jnp.concatenate([b_i, b_c, b_o]).reshape(1, 3 * OUT)
    h_new, c_new = _tc_gates(xp, agg2, hist, wl3, wr3, b3)
    return (h_new[:N], c_new[:N])


# speedup vs baseline: 10.1567x; 1.0891x over previous
import jax
import jax.numpy as jnp
from jax import lax
from jax.experimental import pallas as pl
from jax.experimental.pallas import tpu as pltpu
from jax.experimental.pallas import tpu_sc as plsc

N = 10000
E = 320000
D = 128
OUT = 128

NC = 2
NS = 16
NW = NC * NS

EDGES_PER_W = E // NW
CHUNK = 128
FULL_CHUNKS = EDGES_PER_W // CHUNK
TAIL = EDGES_PER_W - FULL_CHUNKS * CHUNK
NP = 10240
ROWS_PER_TILE = NP // NS
HR = NP // 128


NPAIRS = FULL_CHUNKS // 2


def _sc_body(x_hbm, src_hbm, dst_hbm, zrow_hbm,
             agg_hbm, hist_hbm,
             srcA2, dstA2, rowsA, srcB2, dstB2, rowsB,
             src_t, dst_t, rows_t, hist_v,
             acc_sh,
             semi, semgA, semgB, semsA, semsB):
    c = lax.axis_index("c")
    s = lax.axis_index("s")
    w = c * NS + s
    base = w * EDGES_PER_W
    row0 = s * ROWS_PER_TILE

    pltpu.sync_copy(zrow_hbm, rowsA)
    zd = []
    for k in range(ROWS_PER_TILE // CHUNK):
        zd.append(pltpu.async_copy(
            rowsA, acc_sh.at[pl.ds(row0 + k * CHUNK, CHUNK), :], semsA))

    zeros16 = jnp.zeros((16,), jnp.float32)

    def zstep(i, carry):
        for g in range(128 // 16):
            hist_v[i, pl.ds(g * 16, 16)] = zeros16
        return carry

    lax.fori_loop(0, HR, zstep, 0)
    for d in zd:
        d.wait()
    plsc.subcore_barrier()

    ones16 = jnp.ones((16,), jnp.float32)

    def hist_update(dref):
        for g in range(CHUNK // 16):
            ig = dref[pl.ds(g * 16, 16)]
            plsc.addupdate_scatter(
                hist_v, [lax.shift_right_logical(ig, 7),
                         lax.bitwise_and(ig, 127)], ones16)

    def load_pair_idx(i, sl):
        offA = base + (2 * i) * CHUNK
        offB = offA + CHUNK
        pltpu.async_copy(src_hbm.at[pl.ds(offA, CHUNK)], srcA2.at[sl], semi)
        pltpu.async_copy(dst_hbm.at[pl.ds(offA, CHUNK)], dstA2.at[sl], semi)
        pltpu.async_copy(src_hbm.at[pl.ds(offB, CHUNK)], srcB2.at[sl], semi)
        pltpu.async_copy(dst_hbm.at[pl.ds(offB, CHUNK)], dstB2.at[sl], semi)

    def drain_pair_idx(sl):
        pltpu.make_async_copy(src_hbm.at[pl.ds(base, CHUNK)], srcA2.at[sl], semi).wait()
        pltpu.make_async_copy(dst_hbm.at[pl.ds(base, CHUNK)], dstA2.at[sl], semi).wait()
        pltpu.make_async_copy(src_hbm.at[pl.ds(base, CHUNK)], srcB2.at[sl], semi).wait()
        pltpu.make_async_copy(dst_hbm.at[pl.ds(base, CHUNK)], dstB2.at[sl], semi).wait()

    load_pair_idx(0, 0)

    def step(i, carry):
        sl = lax.bitwise_and(i, 1)
        nsl = 1 - sl
        drain_pair_idx(sl)
        nxt = lax.rem(i + 1, NPAIRS)
        load_pair_idx(nxt, nsl)
        ga = pltpu.async_copy(x_hbm.at[srcA2.at[sl]], rowsA, semgA)
        gb = pltpu.async_copy(x_hbm.at[srcB2.at[sl]], rowsB, semgB)
        ga.wait()
        sa = pltpu.async_copy(rowsA, acc_sh.at[dstA2.at[sl]], semsA, add=True)
        hist_update(dstA2.at[sl])
        gb.wait()
        sb = pltpu.async_copy(rowsB, acc_sh.at[dstB2.at[sl]], semsB, add=True)
        hist_update(dstB2.at[sl])
        sa.wait()
        sb.wait()
        return carry

    lax.fori_loop(0, NPAIRS, step, 0)
    drain_pair_idx(0)

    off = base + FULL_CHUNKS * CHUNK
    pltpu.sync_copy(src_hbm.at[pl.ds(off, TAIL)], src_t)
    pltpu.sync_copy(dst_hbm.at[pl.ds(off, TAIL)], dst_t)
    pltpu.sync_copy(x_hbm.at[src_t], rows_t)
    pltpu.sync_copy(rows_t, acc_sh.at[dst_t], add=True)
    it = dst_t[...]
    plsc.addupdate_scatter(
        hist_v, [lax.shift_right_logical(it, 7),
                 lax.bitwise_and(it, 127)], ones16)

    plsc.subcore_barrier()

    hd = pltpu.async_copy(hist_v, hist_hbm.at[w], semi)
    nk = ROWS_PER_TILE // CHUNK
    bufs = [rowsA, rowsB]
    isems = [semgA, semgB]
    osems = [semsA, semsB]
    din = {0: pltpu.async_copy(acc_sh.at[pl.ds(row0, CHUNK), :], bufs[0],
                               isems[0])}
    dout = {}
    for k in range(nk):
        b = k % 2
        din[k].wait()
        if k + 1 < nk:
            if k + 1 >= 2:
                dout[k - 1].wait()
            din[k + 1] = pltpu.async_copy(
                acc_sh.at[pl.ds(row0 + (k + 1) * CHUNK, CHUNK), :],
                bufs[(k + 1) % 2], isems[(k + 1) % 2])
        dout[k] = pltpu.async_copy(
            bufs[b], agg_hbm.at[c, pl.ds(row0 + k * CHUNK, CHUNK), :],
            osems[b])
    dout[nk - 2].wait()
    dout[nk - 1].wait()
    hd.wait()


@jax.jit
def _sc_segment_sum(x, src, dst):
    zrow = jnp.zeros((CHUNK, D), jnp.float32)
    mesh = plsc.VectorSubcoreMesh(core_axis_name="c", subcore_axis_name="s",
                                  num_cores=NC, num_subcores=NS)
    fn = pl.kernel(
        _sc_body,
        out_type=(jax.ShapeDtypeStruct((NC, NP, D), jnp.float32),
                  jax.ShapeDtypeStruct((NW, HR, 128), jnp.float32)),
        mesh=mesh,
        compiler_params=pltpu.CompilerParams(needs_layout_passes=False),
        scratch_types=[
            pltpu.VMEM((2, CHUNK), jnp.int32),
            pltpu.VMEM((2, CHUNK), jnp.int32),
            pltpu.VMEM((CHUNK, D), jnp.float32),
            pltpu.VMEM((2, CHUNK), jnp.int32),
            pltpu.VMEM((2, CHUNK), jnp.int32),
            pltpu.VMEM((CHUNK, D), jnp.float32),
            pltpu.VMEM((TAIL,), jnp.int32),
            pltpu.VMEM((TAIL,), jnp.int32),
            pltpu.VMEM((TAIL, D), jnp.float32),
            pltpu.VMEM((HR, 128), jnp.float32),
            pltpu.VMEM_SHARED((NP, D), jnp.float32),
            pltpu.SemaphoreType.DMA,
            pltpu.SemaphoreType.DMA,
            pltpu.SemaphoreType.DMA,
            pltpu.SemaphoreType.DMA,
            pltpu.SemaphoreType.DMA,
        ],
    )
    return fn(x, src, dst, zrow)


def _tc_body(x_ref, a_ref, h_ref, wl_ref, wr_ref, b_ref, h_out, c_out):
    cnt_row = jnp.sum(h_ref[...], axis=0, keepdims=True)
    inv_row = 1.0 / jnp.maximum(cnt_row, 1.0)
    inv_col = lax.dot_general(inv_row, jnp.ones((1, D), jnp.float32),
                              ((( 0,), (0,)), ((), ())),
                              precision=lax.Precision.HIGHEST)
    agg = (a_ref[0] + a_ref[1]) * inv_col
    z = (jnp.dot(agg, wl_ref[...], preferred_element_type=jnp.float32,
                 precision=lax.Precision.HIGHEST)
         + jnp.dot(x_ref[...], wr_ref[...], preferred_element_type=jnp.float32,
                   precision=lax.Precision.HIGHEST)
         + b_ref[...])
    i_g = jax.nn.sigmoid(z[:, :OUT])
    t_g = jnp.tanh(z[:, OUT:2 * OUT])
    o_g = jax.nn.sigmoid(z[:, 2 * OUT:])
    c_new = i_g * t_g
    h_out[...] = o_g * jnp.tanh(c_new)
    c_out[...] = c_new


BLK = 1024
HRB = BLK // 128


@jax.jit
def _tc_gates(xp, agg2, hist, wl3, wr3, b3):
    grid = (NP // BLK,)
    return pl.pallas_call(
        _tc_body,
        grid=grid,
        in_specs=[
            pl.BlockSpec((BLK, D), lambda i: (i, 0)),
            pl.BlockSpec((NC, BLK, D), lambda i: (0, i, 0)),
            pl.BlockSpec((NW, BLK), lambda i: (0, i)),
            pl.BlockSpec((D, 3 * OUT), lambda i: (0, 0)),
            pl.BlockSpec((D, 3 * OUT), lambda i: (0, 0)),
            pl.BlockSpec((1, 3 * OUT), lambda i: (0, 0)),
        ],
        out_specs=[
            pl.BlockSpec((BLK, OUT), lambda i: (i, 0)),
            pl.BlockSpec((BLK, OUT), lambda i: (i, 0)),
        ],
        out_shape=[
            jax.ShapeDtypeStruct((NP, OUT), jnp.float32),
            jax.ShapeDtypeStruct((NP, OUT), jnp.float32),
        ],
    )(xp, agg2, hist, wl3, wr3, b3)


def kernel(x, edge_index, Wl_i, Wr_i, b_i, Wl_f, Wr_f, b_f,
           Wl_c, Wr_c, b_c, Wl_o, Wr_o, b_o):
    agg2, hist = _sc_segment_sum(x, edge_index[0], edge_index[1])
    hist = hist.reshape(NW, NP)
    xp = jnp.concatenate([x, jnp.zeros((NP - N, D), x.dtype)], axis=0)
    wl3 = jnp.concatenate([Wl_i, Wl_c, Wl_o], axis=1)
    wr3 = jnp.concatenate([Wr_i, Wr_c, Wr_o], axis=1)
    b3 = jnp.concatenate([b_i, b_c, b_o]).reshape(1, 3 * OUT)
    h_new, c_new = _tc_gates(xp, agg2, hist, wl3, wr3, b3)
    return (h_new[:N], c_new[:N])


# speedup vs baseline: 11.5236x; 1.1346x over previous
import jax
import jax.numpy as jnp
from jax import lax
from jax.experimental import pallas as pl
from jax.experimental.pallas import tpu as pltpu
from jax.experimental.pallas import tpu_sc as plsc

N = 10000
E = 320000
D = 128
OUT = 128

NC = 2
NS = 16
NW = NC * NS

EDGES_PER_W = E // NW
CHUNK = 128
FULL_CHUNKS = EDGES_PER_W // CHUNK
TAIL = EDGES_PER_W - FULL_CHUNKS * CHUNK
NP = 10240
ROWS_PER_TILE = NP // NS
HR = NP // 128


NPAIRS = FULL_CHUNKS // 2


def _sc_body(x_hbm, src_hbm, dst_hbm, zrow_hbm,
             agg_hbm, hist_hbm,
             srcA2, dstA2, rowsA, srcB2, dstB2, rowsB,
             src_t, dst_t, rows_t, hist_v,
             acc_sh,
             semi, semgA, semgB, semsA, semsB):
    c = lax.axis_index("c")
    s = lax.axis_index("s")
    w = c * NS + s
    base = w * EDGES_PER_W
    row0 = s * ROWS_PER_TILE

    pltpu.sync_copy(zrow_hbm, rowsA)
    zd = []
    for k in range(ROWS_PER_TILE // CHUNK):
        zd.append(pltpu.async_copy(
            rowsA, acc_sh.at[pl.ds(row0 + k * CHUNK, CHUNK), :], semsA))

    zeros16 = jnp.zeros((16,), jnp.float32)

    def zstep(i, carry):
        for g in range(128 // 16):
            hist_v[i, pl.ds(g * 16, 16)] = zeros16
        return carry

    lax.fori_loop(0, HR, zstep, 0)
    for d in zd:
        d.wait()
    plsc.subcore_barrier()

    ones16 = jnp.ones((16,), jnp.float32)

    def hist_update(dref):
        for g in range(CHUNK // 16):
            ig = dref[pl.ds(g * 16, 16)]
            plsc.addupdate_scatter(
                hist_v, [lax.shift_right_logical(ig, 7),
                         lax.bitwise_and(ig, 127)], ones16)

    def load_pair_idx(i, sl):
        offA = base + (2 * i) * CHUNK
        offB = offA + CHUNK
        pltpu.async_copy(src_hbm.at[pl.ds(offA, CHUNK)], srcA2.at[sl], semi)
        pltpu.async_copy(dst_hbm.at[pl.ds(offA, CHUNK)], dstA2.at[sl], semi)
        pltpu.async_copy(src_hbm.at[pl.ds(offB, CHUNK)], srcB2.at[sl], semi)
        pltpu.async_copy(dst_hbm.at[pl.ds(offB, CHUNK)], dstB2.at[sl], semi)

    def drain_pair_idx(sl):
        pltpu.make_async_copy(src_hbm.at[pl.ds(base, CHUNK)], srcA2.at[sl], semi).wait()
        pltpu.make_async_copy(dst_hbm.at[pl.ds(base, CHUNK)], dstA2.at[sl], semi).wait()
        pltpu.make_async_copy(src_hbm.at[pl.ds(base, CHUNK)], srcB2.at[sl], semi).wait()
        pltpu.make_async_copy(dst_hbm.at[pl.ds(base, CHUNK)], dstB2.at[sl], semi).wait()

    load_pair_idx(0, 0)

    def step(i, carry):
        sl = lax.bitwise_and(i, 1)
        nsl = 1 - sl
        drain_pair_idx(sl)
        nxt = lax.rem(i + 1, NPAIRS)
        load_pair_idx(nxt, nsl)
        ga = pltpu.async_copy(x_hbm.at[srcA2.at[sl]], rowsA, semgA)
        gb = pltpu.async_copy(x_hbm.at[srcB2.at[sl]], rowsB, semgB)
        ga.wait()
        sa = pltpu.async_copy(rowsA, acc_sh.at[dstA2.at[sl]], semsA, add=True)
        hist_update(dstA2.at[sl])
        gb.wait()
        sb = pltpu.async_copy(rowsB, acc_sh.at[dstB2.at[sl]], semsB, add=True)
        hist_update(dstB2.at[sl])
        sa.wait()
        sb.wait()
        return carry

    lax.fori_loop(0, NPAIRS, step, 0)
    drain_pair_idx(0)

    off = base + FULL_CHUNKS * CHUNK
    pltpu.sync_copy(src_hbm.at[pl.ds(off, TAIL)], src_t)
    pltpu.sync_copy(dst_hbm.at[pl.ds(off, TAIL)], dst_t)
    pltpu.sync_copy(x_hbm.at[src_t], rows_t)
    pltpu.sync_copy(rows_t, acc_sh.at[dst_t], add=True)
    it = dst_t[...]
    plsc.addupdate_scatter(
        hist_v, [lax.shift_right_logical(it, 7),
                 lax.bitwise_and(it, 127)], ones16)

    plsc.subcore_barrier()

    hd = pltpu.async_copy(hist_v, hist_hbm.at[w], semi)
    nk = ROWS_PER_TILE // CHUNK
    bufs = [rowsA, rowsB]
    isems = [semgA, semgB]
    osems = [semsA, semsB]
    din = {0: pltpu.async_copy(acc_sh.at[pl.ds(row0, CHUNK), :], bufs[0],
                               isems[0])}
    dout = {}
    for k in range(nk):
        b = k % 2
        din[k].wait()
        if k + 1 < nk:
            if k + 1 >= 2:
                dout[k - 1].wait()
            din[k + 1] = pltpu.async_copy(
                acc_sh.at[pl.ds(row0 + (k + 1) * CHUNK, CHUNK), :],
                bufs[(k + 1) % 2], isems[(k + 1) % 2])
        dout[k] = pltpu.async_copy(
            bufs[b], agg_hbm.at[c, pl.ds(row0 + k * CHUNK, CHUNK), :],
            osems[b])
    dout[nk - 2].wait()
    dout[nk - 1].wait()
    hd.wait()


@jax.jit
def _sc_segment_sum(x, src, dst):
    zrow = jnp.zeros((CHUNK, D), jnp.float32)
    mesh = plsc.VectorSubcoreMesh(core_axis_name="c", subcore_axis_name="s",
                                  num_cores=NC, num_subcores=NS)
    fn = pl.kernel(
        _sc_body,
        out_type=(jax.ShapeDtypeStruct((NC, NP, D), jnp.float32),
                  jax.ShapeDtypeStruct((NW, HR, 128), jnp.float32)),
        mesh=mesh,
        compiler_params=pltpu.CompilerParams(needs_layout_passes=False),
        scratch_types=[
            pltpu.VMEM((2, CHUNK), jnp.int32),
            pltpu.VMEM((2, CHUNK), jnp.int32),
            pltpu.VMEM((CHUNK, D), jnp.float32),
            pltpu.VMEM((2, CHUNK), jnp.int32),
            pltpu.VMEM((2, CHUNK), jnp.int32),
            pltpu.VMEM((CHUNK, D), jnp.float32),
            pltpu.VMEM((TAIL,), jnp.int32),
            pltpu.VMEM((TAIL,), jnp.int32),
            pltpu.VMEM((TAIL, D), jnp.float32),
            pltpu.VMEM((HR, 128), jnp.float32),
            pltpu.VMEM_SHARED((NP, D), jnp.float32),
            pltpu.SemaphoreType.DMA,
            pltpu.SemaphoreType.DMA,
            pltpu.SemaphoreType.DMA,
            pltpu.SemaphoreType.DMA,
            pltpu.SemaphoreType.DMA,
        ],
    )
    return fn(x, src, dst, zrow)


def _tc_body(x_ref, a_ref, h_ref, wl_ref, wr_ref, b_ref, h_out, c_out):
    cnt_row = jnp.sum(h_ref[...], axis=0, keepdims=True)
    inv_row = 1.0 / jnp.maximum(cnt_row, 1.0)
    inv_col = lax.dot_general(inv_row, jnp.ones((1, D), jnp.float32),
                              ((( 0,), (0,)), ((), ())),
                              precision=lax.Precision.HIGHEST)
    agg = (a_ref[0] + a_ref[1]) * inv_col
    z = (jnp.dot(agg, wl_ref[...], preferred_element_type=jnp.float32)
         + jnp.dot(x_ref[...], wr_ref[...], preferred_element_type=jnp.float32)
         + b_ref[...])
    i_g = jax.nn.sigmoid(z[:, :OUT])
    t_g = jnp.tanh(z[:, OUT:2 * OUT])
    o_g = jax.nn.sigmoid(z[:, 2 * OUT:])
    c_new = i_g * t_g
    h_out[...] = o_g * jnp.tanh(c_new)
    c_out[...] = c_new


BLK = 1024
HRB = BLK // 128


@jax.jit
def _tc_gates(x, agg2, hist, wl3, wr3, b3):
    grid = (NP // BLK,)
    return pl.pallas_call(
        _tc_body,
        grid=grid,
        in_specs=[
            pl.BlockSpec((BLK, D), lambda i: (i, 0)),
            pl.BlockSpec((NC, BLK, D), lambda i: (0, i, 0)),
            pl.BlockSpec((NW, BLK), lambda i: (0, i)),
            pl.BlockSpec((D, 3 * OUT), lambda i: (0, 0)),
            pl.BlockSpec((D, 3 * OUT), lambda i: (0, 0)),
            pl.BlockSpec((1, 3 * OUT), lambda i: (0, 0)),
        ],
        out_specs=[
            pl.BlockSpec((BLK, OUT), lambda i: (i, 0)),
            pl.BlockSpec((BLK, OUT), lambda i: (i, 0)),
        ],
        out_shape=[
            jax.ShapeDtypeStruct((N, OUT), jnp.float32),
            jax.ShapeDtypeStruct((N, OUT), jnp.float32),
        ],
    )(x, agg2, hist, wl3, wr3, b3)


def kernel(x, edge_index, Wl_i, Wr_i, b_i, Wl_f, Wr_f, b_f,
           Wl_c, Wr_c, b_c, Wl_o, Wr_o, b_o):
    agg2, hist = _sc_segment_sum(x, edge_index[0], edge_index[1])
    hist = hist.reshape(NW, NP)
    wl3 = jnp.concatenate([Wl_i, Wl_c, Wl_o], axis=1)
    wr3 = jnp.concatenate([Wr_i, Wr_c, Wr_o], axis=1)
    b3 = jnp.concatenate([b_i, b_c, b_o]).reshape(1, 3 * OUT)
    h_new, c_new = _tc_gates(x, agg2, hist, wl3, wr3, b3)
    return (h_new, c_new)
